# Initial kernel scaffold; baseline (speedup 1.0000x reference)
import jax
import jax.numpy as jnp
from jax import lax
from jax.experimental import pallas as pl
from jax.experimental.pallas import tpu as pltpu
from jax.experimental.pallas import tpu_sc as plsc

N = 10000
E = 320000
D = 128

NC = 2
NS = 16
L = 16
NW = NC * NS
EPT = E // NW
CH = 80
NCHUNK = EPT // CH
GROUPS = CH // L
RPT = N // NS


def _mm_body(x_ref, w_ref, h_ref):
    h_ref[...] = jnp.dot(x_ref[...], w_ref[...],
                         preferred_element_type=jnp.float32)


def _matmul(x, w):
    return pl.pallas_call(
        _mm_body,
        out_shape=jax.ShapeDtypeStruct((N, D), jnp.float32),
    )(x, w)


def _combine_body(acc_ref, h_ref, deg_ref, b_ref, o_ref):
    deg = deg_ref[...] + 1.0
    inv = 1.0 / deg
    o_ref[...] = jnp.maximum(
        acc_ref[0] + acc_ref[1] + inv * h_ref[...] + b_ref[...], 0.0)


def _combine(acc_parts, h, deg2, b2):
    return pl.pallas_call(
        _combine_body,
        out_shape=jax.ShapeDtypeStruct((N, D), jnp.float32),
    )(acc_parts, h, deg2, b2)


def _sc_body(h_hbm, row_hbm, col_hbm, ew_hbm,
             acc_hbm, deg_hbm,
             dis_v, idxr, idxc, ew_v, buf_a, buf_b, zrow, zdeg,
             deg_sh, acc_sh, sem_a, sem_b):
    cid = lax.axis_index("c")
    sid = lax.axis_index("s")
    wid = cid * NS + sid

    z16 = jnp.zeros((L,), jnp.float32)

    @pl.loop(0, 125)
    def _(r):
        for q in range(D // L):
            zrow[r, pl.ds(q * L, L)] = z16

    @pl.loop(0, 2000 // L)
    def _(r):
        zdeg[pl.ds(r * L, L)] = z16

    for t in range(5):
        pltpu.sync_copy(zrow, acc_sh.at[pl.ds(sid * RPT + t * 125, 125)])

    @pl.when(sid == 0)
    def _():
        for t in range(5):
            pltpu.sync_copy(zdeg, deg_sh.at[pl.ds(t * 2000, 2000)])

    plsc.subcore_barrier()

    for half in range(2):
        pltpu.sync_copy(col_hbm.at[2 * sid + half], idxc)
        pltpu.sync_copy(ew_hbm.at[2 * sid + half], ew_v)

        @pl.loop(0, NCHUNK)
        def _(c):
            pltpu.sync_copy(ew_v.at[c], deg_sh.at[idxc.at[c]], add=True)

    plsc.subcore_barrier()

    pltpu.sync_copy(deg_sh, dis_v)

    @pl.loop(0, N // L)
    def _(r):
        sl = pl.ds(r * L, L)
        d = dis_v[sl] + 1.0
        i = plsc.bitcast(d, jnp.int32)
        i = 0x5F3759DF - lax.shift_right_arithmetic(i, 1)
        y = plsc.bitcast(i, jnp.float32)
        for _ in range(3):
            y = y * (1.5 - 0.5 * d * y * y)
        dis_v[sl] = y

    @pl.when(wid == 0)
    def _():
        pltpu.sync_copy(deg_sh, deg_hbm)

    pltpu.sync_copy(row_hbm.at[wid], idxr)
    pltpu.sync_copy(col_hbm.at[wid], idxc)
    pltpu.sync_copy(ew_hbm.at[wid], ew_v)

    def start_gather(c, buf, sem):
        pltpu.make_async_copy(h_hbm.at[idxr.at[c]], buf, sem).start()

    def wait_gather(c, buf, sem):
        pltpu.make_async_copy(h_hbm.at[idxr.at[c]], buf, sem).wait()

    def process(c, buf):
        @pl.loop(0, GROUPS)
        def _(g):
            e0 = g * L
            rv = idxr[c, pl.ds(e0, L)]
            cv = idxc[c, pl.ds(e0, L)]
            ev = ew_v[c, pl.ds(e0, L)]
            s = plsc.load_gather(dis_v, [rv]) * ev * plsc.load_gather(dis_v, [cv])
            for j in range(L):
                sj = s[j]
                for q in range(D // L):
                    slq = pl.ds(q * L, L)
                    buf[e0 + j, slq] = buf[e0 + j, slq] * sj

        pltpu.sync_copy(buf, acc_sh.at[idxc.at[c]], add=True)

    start_gather(0, buf_a, sem_a)

    @pl.loop(0, NCHUNK // 2)
    def _(i):
        c0 = 2 * i
        start_gather(c0 + 1, buf_b, sem_b)
        wait_gather(c0, buf_a, sem_a)
        process(c0, buf_a)
        start_gather(c0 + 2, buf_a, sem_a)
        wait_gather(c0 + 1, buf_b, sem_b)
        process(c0 + 1, buf_b)

    wait_gather(NCHUNK - 1, buf_a, sem_a)
    process(NCHUNK - 1, buf_a)

    plsc.subcore_barrier()

    sl = pl.ds(sid * RPT, RPT)
    pltpu.sync_copy(acc_sh.at[sl], acc_hbm.at[cid, sl])


@jax.jit
def _sc_aggregate(h, row3, col3, ew3):
    mesh = plsc.VectorSubcoreMesh(core_axis_name="c", subcore_axis_name="s")
    fn = pl.kernel(
        _sc_body,
        out_type=(jax.ShapeDtypeStruct((NC, N, D), jnp.float32),
                  jax.ShapeDtypeStruct((N,), jnp.float32)),
        mesh=mesh,
        scratch_types=[
            pltpu.VMEM((N,), jnp.float32),
            pltpu.VMEM((NCHUNK, CH), jnp.int32),
            pltpu.VMEM((NCHUNK, CH), jnp.int32),
            pltpu.VMEM((NCHUNK, CH), jnp.float32),
            pltpu.VMEM((CH, D), jnp.float32),
            pltpu.VMEM((CH, D), jnp.float32),
            pltpu.VMEM((125, D), jnp.float32),
            pltpu.VMEM((2000,), jnp.float32),
            pltpu.VMEM_SHARED((N,), jnp.float32),
            pltpu.VMEM_SHARED((N, D), jnp.float32),
            pltpu.SemaphoreType.DMA,
            pltpu.SemaphoreType.DMA,
        ],
    )
    return fn(h, row3, col3, ew3)


def kernel(x, edge_index, edge_attr, W, b):
    h = _matmul(x, W)
    row3 = edge_index[0].reshape(NW, NCHUNK, CH)
    col3 = edge_index[1].reshape(NW, NCHUNK, CH)
    ew3 = edge_attr.reshape(NW, NCHUNK, CH)
    acc_parts, deg = _sc_aggregate(h, row3, col3, ew3)
    return _combine(acc_parts, h, deg.reshape(N, 1), b.reshape(1, D))


# speedup vs baseline: 29.8220x; 29.8220x over previous
import jax
import jax.numpy as jnp
from jax import lax
from jax.experimental import pallas as pl
from jax.experimental.pallas import tpu as pltpu
from jax.experimental.pallas import tpu_sc as plsc

N = 10000
E = 320000
D = 128

NC = 2
NS = 16
L = 16
NW = NC * NS
EPT = E // NW
CH = 80
NCHUNK = EPT // CH
NPASS = 5
PCH = NCHUNK // NPASS
GROUPS = CH // L
RPT = 624


def _mm_body(x_ref, w_ref, h_ref):
    h_ref[...] = jnp.dot(x_ref[...], w_ref[...],
                         preferred_element_type=jnp.float32)


def _matmul(x, w):
    return pl.pallas_call(
        _mm_body,
        out_shape=jax.ShapeDtypeStruct((N, D), jnp.float32),
    )(x, w)


def _combine_body(acc_ref, h_ref, deg_ref, b_ref, o_ref):
    deg = deg_ref[...] + 1.0
    inv = 1.0 / deg
    o_ref[...] = jnp.maximum(
        acc_ref[0] + acc_ref[1] + inv * h_ref[...] + b_ref[...], 0.0)


def _combine(acc_parts, h, deg2, b2):
    return pl.pallas_call(
        _combine_body,
        out_shape=jax.ShapeDtypeStruct((N, D), jnp.float32),
    )(acc_parts, h, deg2, b2)


def _sc_body(h_hbm, row_hbm, col_hbm, ew_hbm,
             acc_hbm, deg_hbm,
             dis_v, idxr, idxc, ew_v, buf_a, buf_b, zdeg,
             deg_sh, acc_sh, sem_a, sem_b):
    cid = lax.axis_index("c")
    sid = lax.axis_index("s")
    wid = cid * NS + sid

    z16 = jnp.zeros((L,), jnp.float32)

    @pl.loop(0, CH)
    def _(r):
        for q in range(D // L):
            buf_a[r, pl.ds(q * L, L)] = z16

    @pl.loop(0, 2000 // L)
    def _(r):
        zdeg[pl.ds(r * L, L)] = z16

    for t in range(7):
        pltpu.sync_copy(buf_a, acc_sh.at[pl.ds(sid * RPT + t * CH, CH)])
    pltpu.sync_copy(buf_a.at[pl.ds(0, 64)],
                    acc_sh.at[pl.ds(sid * RPT + 7 * CH, 64)])

    @pl.when(sid == NS - 1)
    def _():
        pltpu.sync_copy(buf_a.at[pl.ds(0, 16)], acc_sh.at[pl.ds(NS * RPT, 16)])

    @pl.when(sid == 0)
    def _():
        for t in range(5):
            pltpu.sync_copy(zdeg, deg_sh.at[pl.ds(t * 2000, 2000)])

    plsc.subcore_barrier()

    for half in range(2):
        for p in range(NPASS):
            pltpu.sync_copy(col_hbm.at[2 * sid + half, p], idxc)
            pltpu.sync_copy(ew_hbm.at[2 * sid + half, p], ew_v)

            @pl.loop(0, PCH)
            def _(c):
                pltpu.sync_copy(ew_v.at[c], deg_sh.at[idxc.at[c]], add=True)

    plsc.subcore_barrier()

    pltpu.sync_copy(deg_sh, dis_v)

    @pl.loop(0, N // L)
    def _(r):
        sl = pl.ds(r * L, L)
        d = dis_v[sl] + 1.0
        i = lax.bitcast_convert_type(d, jnp.int32)
        i = 0x5F3759DF - lax.shift_right_arithmetic(i, 1)
        y = lax.bitcast_convert_type(i, jnp.float32)
        for _ in range(3):
            y = y * (1.5 - 0.5 * d * y * y)
        dis_v[sl] = y

    @pl.when(wid == 0)
    def _():
        pltpu.sync_copy(deg_sh, deg_hbm)

    def start_gather(c, buf, sem):
        pltpu.make_async_copy(h_hbm.at[idxr.at[c]], buf, sem).start()

    def wait_gather(c, buf, sem):
        pltpu.make_async_copy(h_hbm.at[idxr.at[c]], buf, sem).wait()

    def process(c, buf):
        @pl.loop(0, GROUPS)
        def _(g):
            e0 = g * L
            rv = idxr[c, pl.ds(e0, L)]
            cv = idxc[c, pl.ds(e0, L)]
            ev = ew_v[c, pl.ds(e0, L)]
            s = plsc.load_gather(dis_v, [rv]) * ev * plsc.load_gather(dis_v, [cv])
            for j in range(L):
                sj = s[j]
                for q in range(D // L):
                    slq = pl.ds(q * L, L)
                    buf[e0 + j, slq] = buf[e0 + j, slq] * sj

        pltpu.sync_copy(buf, acc_sh.at[idxc.at[c]], add=True)

    for p in range(NPASS):
        pltpu.sync_copy(row_hbm.at[wid, p], idxr)
        pltpu.sync_copy(col_hbm.at[wid, p], idxc)
        pltpu.sync_copy(ew_hbm.at[wid, p], ew_v)

        start_gather(0, buf_a, sem_a)

        @pl.loop(0, PCH // 2)
        def _(i):
            c0 = 2 * i
            start_gather(c0 + 1, buf_b, sem_b)
            wait_gather(c0, buf_a, sem_a)
            process(c0, buf_a)
            start_gather(c0 + 2, buf_a, sem_a)
            wait_gather(c0 + 1, buf_b, sem_b)
            process(c0 + 1, buf_b)

        wait_gather(PCH - 1, buf_a, sem_a)
        process(PCH - 1, buf_a)

    plsc.subcore_barrier()

    sl = pl.ds(sid * RPT, RPT)
    pltpu.sync_copy(acc_sh.at[sl], acc_hbm.at[cid, sl])

    @pl.when(sid == NS - 1)
    def _():
        tail = pl.ds(NS * RPT, 16)
        pltpu.sync_copy(acc_sh.at[tail], acc_hbm.at[cid, tail])


@jax.jit
def _sc_aggregate(h, row3, col3, ew3):
    mesh = plsc.VectorSubcoreMesh(core_axis_name="c", subcore_axis_name="s")
    fn = pl.kernel(
        _sc_body,
        out_type=(jax.ShapeDtypeStruct((NC, N, D), jnp.float32),
                  jax.ShapeDtypeStruct((N,), jnp.float32)),
        mesh=mesh,
        compiler_params=pltpu.CompilerParams(needs_layout_passes=False),
        scratch_types=[
            pltpu.VMEM((N,), jnp.float32),
            pltpu.VMEM((PCH, CH), jnp.int32),
            pltpu.VMEM((PCH, CH), jnp.int32),
            pltpu.VMEM((PCH, CH), jnp.float32),
            pltpu.VMEM((CH, D), jnp.float32),
            pltpu.VMEM((CH, D), jnp.float32),
            pltpu.VMEM((2000,), jnp.float32),
            pltpu.VMEM_SHARED((N,), jnp.float32),
            pltpu.VMEM_SHARED((N, D), jnp.float32),
            pltpu.SemaphoreType.DMA,
            pltpu.SemaphoreType.DMA,
        ],
    )
    return fn(h, row3, col3, ew3)


def kernel(x, edge_index, edge_attr, W, b):
    h = _matmul(x, W)
    row3 = edge_index[0].reshape(NW, NPASS, PCH, CH)
    col3 = edge_index[1].reshape(NW, NPASS, PCH, CH)
    ew3 = edge_attr.reshape(NW, NPASS, PCH, CH)
    acc_parts, deg = _sc_aggregate(h, row3, col3, ew3)
    return _combine(acc_parts, h, deg.reshape(N, 1), b.reshape(1, D))
